# baseline (device time: 74473 ns/iter reference)
import jax
import jax.numpy as jnp
from jax import lax
from jax.experimental import pallas as pl
from jax.experimental.pallas import tpu as pltpu

N_DEV = 8
N_PLANE = 4
DELTA = 64


def kernel(x, w_mat, scale_x, scale_w):
    m_per, k = x.shape
    _, n_per = w_mat.shape
    half = m_per // 2
    rest = half - DELTA

    s = (scale_x[0] * scale_w[0]).reshape(1, 1)

    def body(x_ref, w_ref, s_ref, out_ref,
             xc, wbc, zbuf, tr1, br1, tr2, tl1, bl1, bl2,
             br_a, br_b, bl_a, bl_b, czbuf, czb2r, czb2l,
             st_br, st_bl, st_c2r, st_c2l,
             z_send, z_recv, a_send, a_recv,
             br_send, br_recv, bl_send, bl_recv,
             c_send, c_recv, c2_send, c2_recv):
        my = lax.axis_index("i")
        zz = my // N_PLANE
        p = lax.rem(my, N_PLANE)
        right = zz * N_PLANE + lax.rem(p + 1, N_PLANE)
        left = zz * N_PLANE + lax.rem(p + 3, N_PLANE)
        zpartner = lax.rem(my + N_PLANE, N_DEV)
        other = (1 - zz) * N_PLANE

        barrier_sem = pltpu.get_barrier_semaphore()
        for nbr in (left, right, zpartner):
            pl.semaphore_signal(
                barrier_sem, inc=1,
                device_id=(nbr,), device_id_type=pl.DeviceIdType.MESH,
            )
        pl.semaphore_wait(barrier_sem, 3)

        scale = s_ref[0, 0]

        def gemm_store(chunk, row0):
            acc = jnp.dot(
                chunk.astype(jnp.bfloat16), wbc[...],
                preferred_element_type=jnp.float32,
            )
            out_ref[pl.ds(row0, chunk.shape[0]), :] = jnp.maximum(acc * scale, 0.0)

        def rdma(src, dst, send_s, recv_s, target):
            return pltpu.make_async_remote_copy(
                src_ref=src, dst_ref=dst, send_sem=send_s, recv_sem=recv_s,
                device_id=(target,), device_id_type=pl.DeviceIdType.MESH,
            )

        started = []

        def start(rds):
            for r in rds:
                r.start()
            started.extend(rds)
            return rds

        def a_rdma(i, src, dst, target):
            return rdma(src, dst, a_send.at[i], a_recv.at[i], target)

        xc[0] = x_ref[pl.ds(0, half), :].astype(jnp.float8_e4m3fn)
        rd_z0 = rdma(xc.at[0], zbuf.at[0], z_send.at[0], z_recv.at[0], zpartner)
        rd_tr1 = a_rdma(0, xc.at[0], tr1, right)
        rd_tl1 = a_rdma(3, xc.at[0], tl1, left)
        start([rd_z0, rd_tr1, rd_tl1])
        xc[1] = x_ref[pl.ds(half, half), :].astype(jnp.float8_e4m3fn)
        rd_z1 = rdma(xc.at[1], zbuf.at[1], z_send.at[1], z_recv.at[1], zpartner)
        rd_br1 = a_rdma(1, xc.at[1], br1, right)
        rd_bl1 = a_rdma(4, xc.at[1], bl1, left)
        start([rd_z1, rd_br1, rd_bl1])

        wbc[...] = w_ref[...].astype(jnp.bfloat16)
        gemm_store(x_ref[...], my * m_per)

        orig_l1 = zz * N_PLANE + lax.rem(p + 3, N_PLANE)
        orig_r1 = zz * N_PLANE + lax.rem(p + 1, N_PLANE)
        orig_2 = zz * N_PLANE + lax.rem(p + 2, N_PLANE)

        rd_tr1.wait_recv()
        rd_tr2 = start([a_rdma(2, tr1, tr2, right)])[0]
        st_c2r[...] = tr1[pl.ds(0, DELTA), :]
        gemm_store(tr1[...], orig_l1 * m_per)

        rd_tl1.wait_recv()
        gemm_store(tl1[...], orig_r1 * m_per)

        rd_bl1.wait_recv()
        rd_bl2 = start([a_rdma(5, bl1, bl2, left)])[0]
        st_c2l[...] = bl1[pl.ds(rest, DELTA), :]
        gemm_store(bl1[...], orig_r1 * m_per + half)

        rd_br1.wait_recv()
        gemm_store(br1[...], orig_l1 * m_per + half)

        rd_z0.wait_recv()
        rd_z1.wait_recv()
        st_br[...] = zbuf[0, pl.ds(DELTA, rest), :]
        st_bl[...] = zbuf[1, pl.ds(0, rest), :]
        b = start([
            rdma(st_br, br_a, br_send.at[0], br_recv.at[0], right),
            rdma(zbuf.at[1], br_b, br_send.at[1], br_recv.at[1], right),
            rdma(zbuf.at[0], bl_a, bl_send.at[0], bl_recv.at[0], left),
            rdma(st_bl, bl_b, bl_send.at[1], bl_recv.at[1], left),
        ])
        c2 = start([
            rdma(st_c2r, czb2r, c2_send.at[0], c2_recv.at[0], zpartner),
            rdma(st_c2l, czb2l, c2_send.at[1], c2_recv.at[1], zpartner),
        ])
        gemm_store(zbuf[0], zpartner * m_per)
        gemm_store(zbuf[1], zpartner * m_per + half)

        rd_tr2.wait_recv()
        c_r = start([rdma(tr2, czbuf.at[0], c_send.at[0], c_recv.at[0], zpartner)])[0]
        gemm_store(tr2[...], orig_2 * m_per)

        rd_bl2.wait_recv()
        c_l = start([rdma(bl2, czbuf.at[1], c_send.at[1], c_recv.at[1], zpartner)])[0]
        gemm_store(bl2[...], orig_2 * m_per + half)

        orig_br = other + lax.rem(p + 3, N_PLANE)
        orig_bl = other + lax.rem(p + 1, N_PLANE)
        for rd in b:
            rd.wait_recv()
        gemm_store(br_a[...], orig_br * m_per + DELTA)
        gemm_store(br_b[...], orig_br * m_per + half)
        gemm_store(bl_a[...], orig_bl * m_per)
        gemm_store(bl_b[...], orig_bl * m_per + half)

        for rd in c2:
            rd.wait_recv()
        gemm_store(czb2r[...], orig_br * m_per)
        gemm_store(czb2l[...], orig_bl * m_per + half + rest)

        orig_c = other + lax.rem(p + 2, N_PLANE)
        c_r.wait_recv()
        gemm_store(czbuf[0], orig_c * m_per)
        c_l.wait_recv()
        gemm_store(czbuf[1], orig_c * m_per + half)

        for rd in started:
            rd.wait_send()

    f8 = jnp.float8_e4m3fn
    half_buf = lambda: pltpu.VMEM((half, k), f8)
    pair_sem = lambda: pltpu.SemaphoreType.DMA((2,))
    return pl.pallas_call(
        body,
        out_shape=jax.ShapeDtypeStruct((N_DEV * m_per, n_per), jnp.float32),
        in_specs=[
            pl.BlockSpec(memory_space=pltpu.VMEM),
            pl.BlockSpec(memory_space=pltpu.VMEM),
            pl.BlockSpec(memory_space=pltpu.SMEM),
        ],
        out_specs=pl.BlockSpec(memory_space=pltpu.VMEM),
        scratch_shapes=[
            pltpu.VMEM((2, half, k), f8),
            pltpu.VMEM((k, n_per), jnp.bfloat16),
            pltpu.VMEM((2, half, k), f8),
            half_buf(), half_buf(), half_buf(),
            half_buf(), half_buf(), half_buf(),
            pltpu.VMEM((rest, k), f8),
            half_buf(),
            half_buf(),
            pltpu.VMEM((rest, k), f8),
            pltpu.VMEM((2, half, k), f8),
            pltpu.VMEM((DELTA, k), f8),
            pltpu.VMEM((DELTA, k), f8),
            pltpu.VMEM((rest, k), f8),
            pltpu.VMEM((rest, k), f8),
            pltpu.VMEM((DELTA, k), f8),
            pltpu.VMEM((DELTA, k), f8),
            pair_sem(), pair_sem(),
            pltpu.SemaphoreType.DMA((6,)),
            pltpu.SemaphoreType.DMA((6,)),
            pair_sem(), pair_sem(),
            pair_sem(), pair_sem(),
            pair_sem(), pair_sem(),
            pair_sem(), pair_sem(),
        ],
        compiler_params=pltpu.CompilerParams(collective_id=0),
    )(x, w_mat, s)


# device time: 70325 ns/iter; 1.0590x vs baseline; 1.0590x over previous
import jax
import jax.numpy as jnp
from jax import lax
from jax.experimental import pallas as pl
from jax.experimental.pallas import tpu as pltpu

N_DEV = 8
N_PLANE = 4
N_HOP = N_PLANE - 1
DELTA = 64


def kernel(x, w_mat, scale_x, scale_w):
    m_per, k = x.shape
    _, n_per = w_mat.shape
    half = m_per // 2
    rest = half - DELTA

    s = (scale_x[0] * scale_w[0]).reshape(1, 1)

    def body(x_ref, w_ref, s_ref, out_ref,
             xc, wbc, zbuf, comm_ar, comm_al,
             br_a, br_b, bl_a, bl_b, czbuf, czb2r, czb2l,
             st_br, st_bl, st_c2r, st_c2l,
             z_send, z_recv, ar_send, ar_recv, al_send, al_recv,
             br_send, br_recv, bl_send, bl_recv,
             c_send, c_recv, c2_send, c2_recv):
        my = lax.axis_index("i")
        zz = my // N_PLANE
        p = lax.rem(my, N_PLANE)
        right = zz * N_PLANE + lax.rem(p + 1, N_PLANE)
        left = zz * N_PLANE + lax.rem(p + 3, N_PLANE)
        zpartner = lax.rem(my + N_PLANE, N_DEV)
        other = (1 - zz) * N_PLANE

        barrier_sem = pltpu.get_barrier_semaphore()
        for nbr in (left, right, zpartner):
            pl.semaphore_signal(
                barrier_sem, inc=1,
                device_id=(nbr,), device_id_type=pl.DeviceIdType.MESH,
            )
        pl.semaphore_wait(barrier_sem, 3)

        scale = s_ref[0, 0]

        def gemm_store(chunk, row0):
            acc = jnp.dot(
                chunk.astype(jnp.bfloat16), wbc[...],
                preferred_element_type=jnp.float32,
            )
            out_ref[pl.ds(row0, chunk.shape[0]), :] = jnp.maximum(acc * scale, 0.0)

        def rdma(src, dst, send_s, recv_s, target):
            return pltpu.make_async_remote_copy(
                src_ref=src, dst_ref=dst, send_sem=send_s, recv_sem=recv_s,
                device_id=(target,), device_id_type=pl.DeviceIdType.MESH,
            )

        def a_hop(h):
            src_r = xc.at[0] if h == 0 else comm_ar.at[h - 1]
            src_l = xc.at[1] if h == 0 else comm_al.at[h - 1]
            return (
                rdma(src_r, comm_ar.at[h], ar_send.at[h], ar_recv.at[h], right),
                rdma(src_l, comm_al.at[h], al_send.at[h], al_recv.at[h], left),
            )

        def compute_a(h):
            orig_r = zz * N_PLANE + lax.rem(p + (N_PLANE - 1 - h), N_PLANE)
            orig_l = zz * N_PLANE + lax.rem(p + 1 + h, N_PLANE)
            gemm_store(comm_ar[h], orig_r * m_per)
            gemm_store(comm_al[h], orig_l * m_per + half)

        started = []

        def start(rds):
            for r in rds:
                r.start()
            started.extend(rds)
            return rds

        xc[0] = x_ref[pl.ds(0, half), :].astype(jnp.float8_e4m3fn)
        z0 = rdma(xc.at[0], zbuf.at[0], z_send.at[0], z_recv.at[0], zpartner)
        a0r = rdma(xc.at[0], comm_ar.at[0], ar_send.at[0], ar_recv.at[0], right)
        start([z0, a0r])
        xc[1] = x_ref[pl.ds(half, half), :].astype(jnp.float8_e4m3fn)
        z1 = rdma(xc.at[1], zbuf.at[1], z_send.at[1], z_recv.at[1], zpartner)
        a0l = rdma(xc.at[1], comm_al.at[0], al_send.at[0], al_recv.at[0], left)
        start([z1, a0l])

        wbc[...] = w_ref[...].astype(jnp.bfloat16)
        gemm_store(x_ref[...], my * m_per)

        a0r.wait_recv()
        a0l.wait_recv()
        a1 = start(a_hop(1))
        st_c2r[...] = comm_ar[0, pl.ds(0, DELTA), :]
        st_c2l[...] = comm_al[0, pl.ds(rest, DELTA), :]
        compute_a(0)

        z0.wait_recv()
        z1.wait_recv()
        st_br[...] = zbuf[0, pl.ds(DELTA, rest), :]
        st_bl[...] = zbuf[1, pl.ds(0, rest), :]
        b = start([
            rdma(st_br, br_a, br_send.at[0], br_recv.at[0], right),
            rdma(zbuf.at[1], br_b, br_send.at[1], br_recv.at[1], right),
            rdma(zbuf.at[0], bl_a, bl_send.at[0], bl_recv.at[0], left),
            rdma(st_bl, bl_b, bl_send.at[1], bl_recv.at[1], left),
        ])
        c2 = start([
            rdma(st_c2r, czb2r, c2_send.at[0], c2_recv.at[0], zpartner),
            rdma(st_c2l, czb2l, c2_send.at[1], c2_recv.at[1], zpartner),
        ])
        gemm_store(zbuf[0], zpartner * m_per)
        gemm_store(zbuf[1], zpartner * m_per + half)

        for rd in a1:
            rd.wait_recv()
        a2 = start(a_hop(2))
        c = start([
            rdma(comm_ar.at[1], czbuf.at[0], c_send.at[0], c_recv.at[0], zpartner),
            rdma(comm_al.at[1], czbuf.at[1], c_send.at[1], c_recv.at[1], zpartner),
        ])
        compute_a(1)

        orig_br = other + lax.rem(p + 3, N_PLANE)
        orig_bl = other + lax.rem(p + 1, N_PLANE)
        for rd in b:
            rd.wait_recv()
        gemm_store(br_a[...], orig_br * m_per + DELTA)
        gemm_store(br_b[...], orig_br * m_per + half)
        gemm_store(bl_a[...], orig_bl * m_per)
        gemm_store(bl_b[...], orig_bl * m_per + half)

        for rd in c2:
            rd.wait_recv()
        gemm_store(czb2r[...], orig_br * m_per)
        gemm_store(czb2l[...], orig_bl * m_per + half + rest)

        for rd in c:
            rd.wait_recv()
        orig_c = other + lax.rem(p + 2, N_PLANE)
        gemm_store(czbuf[0], orig_c * m_per)
        gemm_store(czbuf[1], orig_c * m_per + half)

        for rd in a2:
            rd.wait_recv()
        compute_a(2)

        for rd in started:
            rd.wait_send()

    f8 = jnp.float8_e4m3fn
    half_pair = lambda: pltpu.VMEM((2, half, k), f8)
    pair_sem = lambda: pltpu.SemaphoreType.DMA((2,))
    hop_sem = lambda: pltpu.SemaphoreType.DMA((N_HOP,))
    return pl.pallas_call(
        body,
        out_shape=jax.ShapeDtypeStruct((N_DEV * m_per, n_per), jnp.float32),
        in_specs=[
            pl.BlockSpec(memory_space=pltpu.VMEM),
            pl.BlockSpec(memory_space=pltpu.VMEM),
            pl.BlockSpec(memory_space=pltpu.SMEM),
        ],
        out_specs=pl.BlockSpec(memory_space=pltpu.VMEM),
        scratch_shapes=[
            half_pair(),
            pltpu.VMEM((k, n_per), jnp.bfloat16),
            half_pair(),
            pltpu.VMEM((N_HOP, half, k), f8),
            pltpu.VMEM((N_HOP, half, k), f8),
            pltpu.VMEM((rest, k), f8),
            pltpu.VMEM((half, k), f8),
            pltpu.VMEM((half, k), f8),
            pltpu.VMEM((rest, k), f8),
            half_pair(),
            pltpu.VMEM((DELTA, k), f8),
            pltpu.VMEM((DELTA, k), f8),
            pltpu.VMEM((rest, k), f8),
            pltpu.VMEM((rest, k), f8),
            pltpu.VMEM((DELTA, k), f8),
            pltpu.VMEM((DELTA, k), f8),
            pair_sem(), pair_sem(),
            hop_sem(), hop_sem(), hop_sem(), hop_sem(),
            pair_sem(), pair_sem(),
            pair_sem(), pair_sem(),
            pair_sem(), pair_sem(),
            pair_sem(), pair_sem(),
        ],
        compiler_params=pltpu.CompilerParams(collective_id=0),
    )(x, w_mat, s)
